# baseline (speedup 1.0000x reference)
import numpy as np

import jax
import jax.numpy as jnp
from jax.experimental import pallas as pl

_N = 65536
_C = 1000
_NBINS = 15
_NS = 4
_RS = 1024

_delta = np.float32(_N) / np.float32(_NBINS)
_xq = np.arange(_NBINS + 1, dtype=np.float32) * _delta
_F = [int(np.floor(float(_xq[i]))) for i in range(1, _NBINS)]
_FRAC = [float(np.float32(float(_xq[i]) - np.floor(float(_xq[i]))))
         for i in range(1, _NBINS)]
_NF = len(_F)

_LO0 = 0x3A800000
_HI0 = 0x3F800000


def _conf_acc_body(*refs):
    logit_refs = refs[:_NS]
    labels_ref = refs[_NS]
    conf_ref = refs[_NS + 1]
    acc_ref = refs[_NS + 2]
    for k in range(_NS):
        x = logit_refs[k][...]
        m = jnp.max(x, axis=1, keepdims=True)
        s = jnp.sum(jnp.exp(x - m), axis=1, keepdims=True)
        colids = jax.lax.broadcasted_iota(jnp.int32, x.shape, 1)
        ismax = x == m
        pred = jnp.min(jnp.where(ismax, colids, jnp.int32(_C)), axis=1,
                       keepdims=True)
        lab = labels_ref[pl.ds(k * _RS, _RS), :]
        conf_ref[pl.ds(k * _RS, _RS), :] = 1.0 / s
        acc_ref[pl.ds(k * _RS, _RS), :] = (pred == lab).astype(jnp.float32)


def _ece_body(conf_ref, acc_ref, out_ref):
    conf = conf_ref[...]
    acc = acc_ref[...]
    bits = jax.lax.bitcast_convert_type(conf, jnp.int32)

    lo = [jnp.int32(_LO0)] * _NF
    hi = [jnp.int32(_HI0)] * _NF
    for _ in range(28):
        for j in range(_NF):
            mid = (lo[j] + hi[j]) >> 1
            cnt = jnp.sum((bits <= mid).astype(jnp.int32))
            take = cnt >= jnp.int32(_F[j] + 1)
            hi[j] = jnp.where(take, mid, hi[j])
            lo[j] = jnp.where(take, lo[j], mid + jnp.int32(1))
    big = jnp.int32(1 << 30)
    nxt = []
    for j in range(_NF):
        cnt = jnp.sum((bits <= lo[j]).astype(jnp.int32))
        nmin = jnp.min(jnp.where(bits > lo[j], bits, big))
        nxt.append(jnp.where(cnt >= jnp.int32(_F[j] + 2), lo[j], nmin))
    vmin = jnp.min(bits)
    vmax = jnp.max(bits)
    vals_i = jnp.stack([vmin] + [x for p in zip(lo, nxt) for x in p]
                       + [vmax])
    vals = jax.lax.bitcast_convert_type(vals_i, jnp.float32)

    b = [None] * (_NBINS + 1)
    b[0] = vals[0]
    for i in range(1, _NBINS):
        vlo = vals[2 * i - 1]
        vhi = vals[2 * i]
        b[i] = vlo + jnp.float32(_FRAC[i - 1]) * (vhi - vlo)
    b[_NBINS] = vals[29]

    ece = jnp.float32(0.0)
    mprev = (conf <= b[0]).astype(jnp.float32)
    cp = jnp.sum(mprev)
    sp = jnp.sum(conf * mprev)
    ap = jnp.sum(acc * mprev)
    for i in range(1, _NBINS + 1):
        mcur = (conf <= b[i]).astype(jnp.float32)
        cc = jnp.sum(mcur)
        sc = jnp.sum(conf * mcur)
        ac = jnp.sum(acc * mcur)
        cnt = cc - cp
        safe = jnp.maximum(cnt, 1.0)
        contrib = jnp.abs((sc - sp) / safe - (ac - ap) / safe) * (cnt / _N)
        ece = ece + jnp.where(cnt > 0, contrib, 0.0)
        cp, sp, ap = cc, sc, ac
    out_ref[...] = jnp.broadcast_to(ece, (1, 1))


def kernel(logits, labels):
    n, c = logits.shape
    grid = n // (_RS * _NS)

    def _stream_spec(k):
        return pl.BlockSpec((_RS, c), lambda i, k=k: (_NS * i + k, 0))

    conf2d, acc2d = pl.pallas_call(
        _conf_acc_body,
        grid=(grid,),
        in_specs=[_stream_spec(k) for k in range(_NS)]
        + [pl.BlockSpec((_RS * _NS, 1), lambda i: (i, 0))],
        out_specs=[
            pl.BlockSpec((_RS * _NS, 1), lambda i: (i, 0)),
            pl.BlockSpec((_RS * _NS, 1), lambda i: (i, 0)),
        ],
        out_shape=[
            jax.ShapeDtypeStruct((n, 1), jnp.float32),
            jax.ShapeDtypeStruct((n, 1), jnp.float32),
        ],
    )(*([logits] * _NS + [labels.reshape(n, 1)]))

    conf = conf2d.reshape(n // 128, 128)
    accv = acc2d.reshape(n // 128, 128)
    out = pl.pallas_call(
        _ece_body,
        in_specs=[
            pl.BlockSpec((n // 128, 128), lambda: (0, 0)),
            pl.BlockSpec((n // 128, 128), lambda: (0, 0)),
        ],
        out_specs=pl.BlockSpec((1, 1), lambda: (0, 0)),
        out_shape=jax.ShapeDtypeStruct((1, 1), jnp.float32),
    )(conf, accv)
    return out.reshape((1,))
